# baseline (speedup 1.0000x reference)
import math

import jax
import jax.numpy as jnp
from jax.experimental import pallas as pl
from jax.experimental.pallas import tpu as pltpu

H = 1024
NH = 16
HD = H // NH
W = 128
TOPK = 64
KSEL = TOPK - 2
T = 2048
NB = T // W


def _dot(a, b, dims):
    return jax.lax.dot_general(a, b, (dims, ((), ())),
                               preferred_element_type=jnp.float32)


def _dotx(a, b, dims):
    return jax.lax.dot_general(a, b, (dims, ((), ())),
                               preferred_element_type=jnp.float32,
                               precision=jax.lax.Precision.HIGHEST)


def _softmax_rows(s):
    m = jnp.max(s, axis=-1, keepdims=True)
    e = jnp.exp(s - m)
    return e / jnp.sum(e, axis=-1, keepdims=True)


def _excl_prefix(x):
    rio = jax.lax.broadcasted_iota(jnp.int32, (W, W), 0)
    cio = jax.lax.broadcasted_iota(jnp.int32, (W, W), 1)
    upper = (rio <= cio).astype(jnp.float32)
    incl = _dot(x, upper, (((1,), (0,))))
    rt = incl[:, W - 1:W]
    a = jax.lax.broadcasted_iota(jnp.int32, (NB, NB), 0)
    b = jax.lax.broadcasted_iota(jnp.int32, (NB, NB), 1)
    lower = (b < a).astype(jnp.float32)
    offs = _dot(lower, rt, (((1,), (0,))))
    return incl - x + offs


def _fused_kernel(x_ref, wq_ref, bq_ref, wk_ref, bk_ref, wv_ref, bv_ref,
                  out_ref, p2d_ref):
    scale = 1.0 / math.sqrt(HD)
    x = x_ref[0]
    qp = _dot(x, wq_ref[...], (((1,), (0,)))) + bq_ref[...]
    kp = _dot(x, wk_ref[...], (((1,), (0,)))) + bk_ref[...]
    vp = _dot(x, wv_ref[...], (((1,), (0,)))) + bv_ref[...]

    for h in range(2):
        q = qp[:, h * HD:(h + 1) * HD]
        k16 = kp[:, h * HD:(h + 1) * HD].astype(jnp.bfloat16)
        v16 = vp[:, h * HD:(h + 1) * HD].astype(jnp.bfloat16)
        q16 = q.astype(jnp.bfloat16)

        q3 = q.reshape(NB, W, HD)
        ns = jnp.sum(q3 * q3, axis=2)
        bits = jax.lax.bitcast_convert_type(ns, jnp.int32)

        def bit_body(i, t):
            cand = t | (jnp.int32(1) << (jnp.int32(30) - i))
            cnt = jnp.sum((bits >= cand).astype(jnp.int32))
            return jnp.where(cnt >= KSEL, cand, t)

        thr = jax.lax.fori_loop(0, 31, bit_body, jnp.int32(0))
        gt = bits > thr
        tie = bits == thr
        need = (KSEL - jnp.sum(gt.astype(jnp.int32))).astype(jnp.float32)
        tie_rank = _excl_prefix(tie.astype(jnp.float32))
        sel = gt | (tie & (tie_rank < need))
        rio = jax.lax.broadcasted_iota(jnp.int32, (NB, W), 0)
        cio = jax.lax.broadcasted_iota(jnp.int32, (NB, W), 1)
        flat = rio * W + cio
        m = sel | (flat == 0) | (flat == T - 1)
        mf = m.astype(jnp.float32)

        em = _excl_prefix(mf).astype(jnp.int32)
        r64 = jax.lax.broadcasted_iota(jnp.int32, (TOPK, NB, W), 0)
        p3 = jnp.where((r64 == em[None]) & m[None], 1.0, 0.0)
        for b in range(NB):
            p2d_ref[:, b * W:(b + 1) * W] = p3[:, b, :]
        p2 = p2d_ref[...]

        qg = _dot(p2, q, (((1,), (0,))))
        gs = _dot(qg.astype(jnp.bfloat16), k16, (((1,), (1,)))) * scale
        gp = _softmax_rows(gs)
        gctx = _dot(gp.astype(jnp.bfloat16), v16, (((1,), (0,))))

        mcol = _dot(p2, jnp.ones((TOPK, 1), jnp.float32), (((0,), (0,))))
        scat = _dotx(p2, gctx, (((0,), (0,))))

        for b in range(NB):
            lo = max(0, (b - 1) * W)
            hi = min(T, (b + 2) * W)
            qb = q16[b * W:(b + 1) * W]
            s = _dot(qb, k16[lo:hi], (((1,), (1,)))) * scale
            p = _softmax_rows(s)
            lb = _dot(p.astype(jnp.bfloat16), v16[lo:hi], (((1,), (0,))))
            sl = slice(b * W, (b + 1) * W)
            out_ref[0, sl, h * HD:(h + 1) * HD] = (
                lb * (1.0 - mcol[sl]) + scat[sl])


def kernel(hidden_states, Wq, bq, Wk, bk, Wv, bv):
    n, t, _ = hidden_states.shape
    wspec = pl.BlockSpec((H, 2 * HD), lambda ni, hp: (0, hp))
    bspec = pl.BlockSpec((1, 2 * HD), lambda ni, hp: (0, hp))

    out = pl.pallas_call(
        _fused_kernel,
        grid=(n, NH // 2),
        in_specs=[
            pl.BlockSpec((1, t, H), lambda ni, hp: (ni, 0, 0)),
            wspec, bspec, wspec, bspec, wspec, bspec,
        ],
        out_specs=pl.BlockSpec((1, t, 2 * HD), lambda ni, hp: (ni, 0, hp)),
        out_shape=jax.ShapeDtypeStruct((n, t, H), jnp.float32),
        scratch_shapes=[pltpu.VMEM((TOPK, T), jnp.float32)],
        compiler_params=pltpu.CompilerParams(
            dimension_semantics=("parallel", "parallel")),
    )(hidden_states, Wq, bq.reshape(1, H), Wk, bk.reshape(1, H),
      Wv, bv.reshape(1, H))
    return out


# speedup vs baseline: 1.1865x; 1.1865x over previous
import math

import jax
import jax.numpy as jnp
from jax.experimental import pallas as pl
from jax.experimental.pallas import tpu as pltpu

H = 1024
NH = 16
HD = H // NH
W = 128
TOPK = 64
KSEL = TOPK - 2
T = 2048
NB = T // W


def _dot(a, b, dims):
    return jax.lax.dot_general(a, b, (dims, ((), ())),
                               preferred_element_type=jnp.float32)


def _dotx(a, b, dims):
    return jax.lax.dot_general(a, b, (dims, ((), ())),
                               preferred_element_type=jnp.float32,
                               precision=jax.lax.Precision.HIGHEST)


def _softmax_rows(s):
    m = jnp.max(s, axis=-1, keepdims=True)
    e = jnp.exp(s - m)
    return e / jnp.sum(e, axis=-1, keepdims=True)


def _excl_prefix(x):
    rio = jax.lax.broadcasted_iota(jnp.int32, (W, W), 0)
    cio = jax.lax.broadcasted_iota(jnp.int32, (W, W), 1)
    upper = (rio <= cio).astype(jnp.float32)
    incl = _dot(x, upper, (((1,), (0,))))
    rt = incl[:, W - 1:W]
    a = jax.lax.broadcasted_iota(jnp.int32, (NB, NB), 0)
    b = jax.lax.broadcasted_iota(jnp.int32, (NB, NB), 1)
    lower = (b < a).astype(jnp.float32)
    offs = _dot(lower, rt, (((1,), (0,))))
    return incl - x + offs


def _select_mask(bits):
    def bit_body(i, t):
        cand = t | (jnp.int32(1) << (jnp.int32(30) - i))
        cnt = jnp.sum((bits >= cand).astype(jnp.int32))
        return jnp.where(cnt >= KSEL, cand, t)

    t = jax.lax.fori_loop(0, 31, bit_body, jnp.int32(0))
    gt = bits > t
    tie = bits == t
    need = (KSEL - jnp.sum(gt.astype(jnp.int32))).astype(jnp.float32)
    tie_rank = _excl_prefix(tie.astype(jnp.float32))
    sel = gt | (tie & (tie_rank < need))
    rio = jax.lax.broadcasted_iota(jnp.int32, (NB, W), 0)
    cio = jax.lax.broadcasted_iota(jnp.int32, (NB, W), 1)
    flat = rio * W + cio
    m = sel | (flat == 0) | (flat == T - 1)
    return m


def _fused_kernel(x_ref, wq_ref, bq_ref, wk_ref, bk_ref, wv_ref, bv_ref,
                  out_ref, p2d_ref):
    scale = 1.0 / math.sqrt(HD)
    x = x_ref[0]
    qp = _dot(x, wq_ref[...], (((1,), (0,)))) + bq_ref[...]
    kp = _dot(x, wk_ref[...], (((1,), (0,)))) + bk_ref[...]
    vp = _dot(x, wv_ref[...], (((1,), (0,)))) + bv_ref[...]

    hs = (0, 1)
    q = [qp[:, h * HD:(h + 1) * HD] for h in hs]
    k16 = [kp[:, h * HD:(h + 1) * HD].astype(jnp.bfloat16) for h in hs]
    v16 = [vp[:, h * HD:(h + 1) * HD].astype(jnp.bfloat16) for h in hs]
    q16 = [q[h].astype(jnp.bfloat16) for h in hs]

    q3 = [q[h].reshape(NB, W, HD) for h in hs]
    ns = [jnp.sum(q3[h] * q3[h], axis=2) for h in hs]
    bits = [jax.lax.bitcast_convert_type(ns[h], jnp.int32) for h in hs]
    m = [_select_mask(bits[h]) for h in hs]

    r64 = jax.lax.broadcasted_iota(jnp.int32, (TOPK, NB, W), 0)
    p2 = []
    for h in hs:
        em = _excl_prefix(m[h].astype(jnp.float32)).astype(jnp.int32)
        p3 = jnp.where((r64 == em[None]) & m[h][None], 1.0, 0.0)
        for b in range(NB):
            p2d_ref[h, :, b * W:(b + 1) * W] = p3[:, b, :]
        p2.append(p2d_ref[h])

    ones_t = jnp.ones((TOPK, 1), jnp.float32)
    qg = [_dot(p2[h], q[h], (((1,), (0,)))) for h in hs]
    gs = [_dot(qg[h].astype(jnp.bfloat16), k16[h], (((1,), (1,)))) * scale
          for h in hs]
    gp = [_softmax_rows(gs[h]) for h in hs]
    gctx = [_dot(gp[h].astype(jnp.bfloat16), v16[h], (((1,), (0,))))
            for h in hs]
    aug = [jnp.concatenate([gctx[h], ones_t], axis=1) for h in hs]
    scat2 = [_dot(p2[h], aug[h], (((0,), (0,)))) for h in hs]
    mcol = [scat2[h][:, HD:HD + 1] for h in hs]
    scat = [scat2[h][:, :HD] for h in hs]

    for b in range(NB):
        lo = max(0, (b - 1) * W)
        hi = min(T, (b + 2) * W)
        sl = slice(b * W, (b + 1) * W)
        s = [_dot(q16[h][sl], k16[h][lo:hi], (((1,), (1,)))) * scale
             for h in hs]
        p = [_softmax_rows(s[h]) for h in hs]
        lb = [_dot(p[h].astype(jnp.bfloat16), v16[h][lo:hi], (((1,), (0,))))
              for h in hs]
        for h in hs:
            out_ref[0, sl, h * HD:(h + 1) * HD] = (
                lb[h] * (1.0 - mcol[h][sl]) + scat[h][sl])


def kernel(hidden_states, Wq, bq, Wk, bk, Wv, bv):
    n, t, _ = hidden_states.shape
    wspec = pl.BlockSpec((H, 2 * HD), lambda ni, hp: (0, hp))
    bspec = pl.BlockSpec((1, 2 * HD), lambda ni, hp: (0, hp))

    out = pl.pallas_call(
        _fused_kernel,
        grid=(n, NH // 2),
        in_specs=[
            pl.BlockSpec((1, t, H), lambda ni, hp: (ni, 0, 0)),
            wspec, bspec, wspec, bspec, wspec, bspec,
        ],
        out_specs=pl.BlockSpec((1, t, 2 * HD), lambda ni, hp: (ni, 0, hp)),
        out_shape=jax.ShapeDtypeStruct((n, t, H), jnp.float32),
        scratch_shapes=[pltpu.VMEM((2, TOPK, T), jnp.float32)],
        compiler_params=pltpu.CompilerParams(
            dimension_semantics=("parallel", "parallel")),
    )(hidden_states, Wq, bq.reshape(1, H), Wk, bk.reshape(1, H),
      Wv, bv.reshape(1, H))
    return out


# speedup vs baseline: 1.1886x; 1.0018x over previous
import math

import jax
import jax.numpy as jnp
from jax.experimental import pallas as pl
from jax.experimental.pallas import tpu as pltpu

H = 1024
NH = 16
HD = H // NH
W = 128
TOPK = 64
KSEL = TOPK - 2
T = 2048
NB = T // W


def _dot(a, b, dims):
    return jax.lax.dot_general(a, b, (dims, ((), ())),
                               preferred_element_type=jnp.float32)


def _dotx(a, b, dims):
    return jax.lax.dot_general(a, b, (dims, ((), ())),
                               preferred_element_type=jnp.float32,
                               precision=jax.lax.Precision.HIGHEST)


def _softmax_rows(s):
    m = jnp.max(s, axis=-1, keepdims=True)
    e = jnp.exp(s - m)
    return e / jnp.sum(e, axis=-1, keepdims=True)


def _excl_prefix(x):
    rio = jax.lax.broadcasted_iota(jnp.int32, (W, W), 0)
    cio = jax.lax.broadcasted_iota(jnp.int32, (W, W), 1)
    upper = (rio <= cio).astype(jnp.float32)
    incl = _dot(x, upper, (((1,), (0,))))
    rt = incl[:, W - 1:W]
    a = jax.lax.broadcasted_iota(jnp.int32, (NB, NB), 0)
    b = jax.lax.broadcasted_iota(jnp.int32, (NB, NB), 1)
    lower = (b < a).astype(jnp.float32)
    offs = _dot(lower, rt, (((1,), (0,))))
    return incl - x + offs


def _thresholds(bits_pair):
    def step(t, bits, p1, p0):
        c1 = t | (jnp.int32(1) << p1)
        c2 = t | (jnp.int32(1) << p0)
        c3 = c1 | c2
        n1 = jnp.sum((bits >= c1).astype(jnp.int32))
        n2 = jnp.sum((bits >= c2).astype(jnp.int32))
        n3 = jnp.sum((bits >= c3).astype(jnp.int32))
        return jnp.where(n3 >= KSEL, c3,
                         jnp.where(n1 >= KSEL, c1,
                                   jnp.where(n2 >= KSEL, c2, t)))

    def body(i, carry):
        t0, t1 = carry
        p1 = jnp.int32(30) - 2 * i
        p0 = p1 - 1
        return (step(t0, bits_pair[0], p1, p0),
                step(t1, bits_pair[1], p1, p0))

    t0, t1 = jax.lax.fori_loop(0, 15, body, (jnp.int32(0), jnp.int32(0)))
    out = []
    for t, bits in ((t0, bits_pair[0]), (t1, bits_pair[1])):
        cand = t | jnp.int32(1)
        cnt = jnp.sum((bits >= cand).astype(jnp.int32))
        out.append(jnp.where(cnt >= KSEL, cand, t))
    return out


def _select_mask(bits, t):
    gt = bits > t
    tie = bits == t
    need = (KSEL - jnp.sum(gt.astype(jnp.int32))).astype(jnp.float32)
    tie_rank = _excl_prefix(tie.astype(jnp.float32))
    sel = gt | (tie & (tie_rank < need))
    rio = jax.lax.broadcasted_iota(jnp.int32, (NB, W), 0)
    cio = jax.lax.broadcasted_iota(jnp.int32, (NB, W), 1)
    flat = rio * W + cio
    m = sel | (flat == 0) | (flat == T - 1)
    return m


def _fused_kernel(x_ref, wq_ref, bq_ref, wk_ref, bk_ref, wv_ref, bv_ref,
                  out_ref, p2d_ref):
    scale = 1.0 / math.sqrt(HD)
    x = x_ref[0]
    qp = _dot(x, wq_ref[...], (((1,), (0,)))) + bq_ref[...]
    kp = _dot(x, wk_ref[...], (((1,), (0,)))) + bk_ref[...]
    vp = _dot(x, wv_ref[...], (((1,), (0,)))) + bv_ref[...]

    hs = (0, 1)
    q = [qp[:, h * HD:(h + 1) * HD] for h in hs]
    k16 = [kp[:, h * HD:(h + 1) * HD].astype(jnp.bfloat16) for h in hs]
    v16 = [vp[:, h * HD:(h + 1) * HD].astype(jnp.bfloat16) for h in hs]
    q16 = [q[h].astype(jnp.bfloat16) for h in hs]

    q3 = [q[h].reshape(NB, W, HD) for h in hs]
    ns = [jnp.sum(q3[h] * q3[h], axis=2) for h in hs]
    bits = [jax.lax.bitcast_convert_type(ns[h], jnp.int32) for h in hs]
    thr = _thresholds(bits)
    m = [_select_mask(bits[h], thr[h]) for h in hs]

    r64 = jax.lax.broadcasted_iota(jnp.int32, (TOPK, NB, W), 0)
    p2 = []
    for h in hs:
        em = _excl_prefix(m[h].astype(jnp.float32)).astype(jnp.int32)
        p3 = jnp.where((r64 == em[None]) & m[h][None], 1.0, 0.0)
        for b in range(NB):
            p2d_ref[h, :, b * W:(b + 1) * W] = p3[:, b, :]
        p2.append(p2d_ref[h])

    ones_t = jnp.ones((TOPK, 1), jnp.float32)
    qg = [_dot(p2[h], q[h], (((1,), (0,)))) for h in hs]
    gs = [_dot(qg[h].astype(jnp.bfloat16), k16[h], (((1,), (1,)))) * scale
          for h in hs]
    gp = [_softmax_rows(gs[h]) for h in hs]
    gctx = [_dot(gp[h].astype(jnp.bfloat16), v16[h], (((1,), (0,))))
            for h in hs]
    aug = [jnp.concatenate([gctx[h], ones_t], axis=1) for h in hs]
    scat2 = [_dot(p2[h], aug[h], (((0,), (0,)))) for h in hs]
    mcol = [scat2[h][:, HD:HD + 1] for h in hs]
    scat = [scat2[h][:, :HD] for h in hs]

    for b in range(NB):
        lo = max(0, (b - 1) * W)
        hi = min(T, (b + 2) * W)
        sl = slice(b * W, (b + 1) * W)
        s = [_dot(q16[h][sl], k16[h][lo:hi], (((1,), (1,)))) * scale
             for h in hs]
        p = [_softmax_rows(s[h]) for h in hs]
        lb = [_dot(p[h].astype(jnp.bfloat16), v16[h][lo:hi], (((1,), (0,))))
              for h in hs]
        for h in hs:
            out_ref[0, sl, h * HD:(h + 1) * HD] = (
                lb[h] * (1.0 - mcol[h][sl]) + scat[h][sl])


def kernel(hidden_states, Wq, bq, Wk, bk, Wv, bv):
    n, t, _ = hidden_states.shape
    wspec = pl.BlockSpec((H, 2 * HD), lambda ni, hp: (0, hp))
    bspec = pl.BlockSpec((1, 2 * HD), lambda ni, hp: (0, hp))

    out = pl.pallas_call(
        _fused_kernel,
        grid=(n, NH // 2),
        in_specs=[
            pl.BlockSpec((1, t, H), lambda ni, hp: (ni, 0, 0)),
            wspec, bspec, wspec, bspec, wspec, bspec,
        ],
        out_specs=pl.BlockSpec((1, t, 2 * HD), lambda ni, hp: (ni, 0, hp)),
        out_shape=jax.ShapeDtypeStruct((n, t, H), jnp.float32),
        scratch_shapes=[pltpu.VMEM((2, TOPK, T), jnp.float32)],
        compiler_params=pltpu.CompilerParams(
            dimension_semantics=("parallel", "parallel")),
    )(hidden_states, Wq, bq.reshape(1, H), Wk, bk.reshape(1, H),
      Wv, bv.reshape(1, H))
    return out
